# Initial kernel scaffold; baseline (speedup 1.0000x reference)
import jax
import jax.numpy as jnp
from jax.experimental import pallas as pl


def kernel(logprobs, beam_logprobs_sum, beam_seq, beam_seq_logprobs, state, beam_size):
    raise NotImplementedError("write your pallas kernel here")


# speedup vs baseline: 36.9375x; 36.9375x over previous
import jax
import jax.numpy as jnp
from jax import lax
from jax.experimental import pallas as pl


def _beam_step_kernel(lp_ref, sums_ref, seq_ref, bsl_ref, st_ref,
                      seq_out_ref, ys_out_ref, bsl_out_ref, st_out_ref):
    K = lp_ref.shape[1]
    V = lp_ref.shape[2]
    T = bsl_ref.shape[2]
    lp = lp_ref[0]
    cand = lp + sums_ref[0]
    iota_r = lax.broadcasted_iota(jnp.int32, (K, V), 0)
    iota_c = lax.broadcasted_iota(jnp.int32, (K, V), 1)
    flat = iota_r * V + iota_c
    i8 = lax.broadcasted_iota(jnp.int32, (1, 8), 1)
    i16 = lax.broadcasted_iota(jnp.int32, (1, 16), 1)
    i10 = lax.broadcasted_iota(jnp.int32, (1, K * T), 1)
    seq_row = seq_ref[0]
    ys_row = jnp.zeros((1, 8), jnp.float32)
    seq_out_row = jnp.zeros((1, 16), jnp.int32)
    for j in range(K):
        m = jnp.max(cand)
        sel = jnp.min(jnp.where(cand == m, flat, jnp.int32(2147483647)))
        bix = sel // V
        six = sel - bix * V
        cand = jnp.where(flat == sel, -jnp.inf, cand)
        ys_row = jnp.where(i8 == j, m, ys_row)
        for t in range(T):
            val = jnp.sum(jnp.where(i10 == bix * T + t, seq_row, 0))
            seq_out_row = jnp.where(i16 == j * (T + 1) + t, val, seq_out_row)
        seq_out_row = jnp.where(i16 == j * (T + 1) + T, six, seq_out_row)
        bsl_out_ref[0, pl.ds(j, 1), pl.ds(0, T), :] = bsl_ref[0, pl.ds(bix, 1), :, :]
        bsl_out_ref[0, pl.ds(j, 1), pl.ds(T, 1), :] = lp_ref[pl.ds(0, 1), pl.ds(bix, 1), :]
        st_out_ref[:, 0, pl.ds(j, 1), :] = st_ref[:, 0, pl.ds(bix, 1), :]
    ys_out_ref[0] = ys_row
    seq_out_ref[0] = seq_out_row


def kernel(logprobs, beam_logprobs_sum, beam_seq, beam_seq_logprobs, state,
           beam_size):
    B, K = beam_logprobs_sum.shape
    V = logprobs.shape[-1]
    T = beam_seq.shape[-1]
    S, BK, D = state.shape

    lp3 = logprobs.reshape(B, K, V)
    sums3 = beam_logprobs_sum.reshape(B, K, 1)
    seq3 = beam_seq.reshape(B, 1, K * T)
    st4 = state.reshape(S, B, K, D)

    out_shapes = (
        jax.ShapeDtypeStruct((B, 1, 16), jnp.int32),
        jax.ShapeDtypeStruct((B, 1, 8), jnp.float32),
        jax.ShapeDtypeStruct((B, K, T + 1, V), jnp.float32),
        jax.ShapeDtypeStruct((S, B, K, D), jnp.float32),
    )
    seq_out, ys_out, bsl_out, st_out = pl.pallas_call(
        _beam_step_kernel,
        grid=(B,),
        in_specs=[
            pl.BlockSpec((1, K, V), lambda b: (b, 0, 0)),
            pl.BlockSpec((1, K, 1), lambda b: (b, 0, 0)),
            pl.BlockSpec((1, 1, K * T), lambda b: (b, 0, 0)),
            pl.BlockSpec((1, K, T, V), lambda b: (b, 0, 0, 0)),
            pl.BlockSpec((S, 1, K, D), lambda b: (0, b, 0, 0)),
        ],
        out_specs=[
            pl.BlockSpec((1, 1, 16), lambda b: (b, 0, 0)),
            pl.BlockSpec((1, 1, 8), lambda b: (b, 0, 0)),
            pl.BlockSpec((1, K, T + 1, V), lambda b: (b, 0, 0, 0)),
            pl.BlockSpec((S, 1, K, D), lambda b: (0, b, 0, 0)),
        ],
        out_shape=out_shapes,
    )(lp3, sums3, seq3, beam_seq_logprobs, st4)

    new_beam_seq = seq_out[:, 0, :K * (T + 1)].reshape(B, K, T + 1)
    new_beam_logprobs_sum = ys_out[:, 0, :K]
    new_state = st_out.reshape(S, B * K, D)
    return (new_beam_seq, bsl_out, new_beam_logprobs_sum, new_state)
